# baseline (speedup 1.0000x reference)
import functools

import jax
import jax.numpy as jnp
from jax import lax
from jax.experimental import pallas as pl
from jax.experimental.pallas import tpu as pltpu
from jax.experimental.pallas import tpu_sc as plsc

N_NODES = 10000
NPAD = 10240
E_EDGES = 320000
NUM_TILES = 32
CHUNK = 128
C_CHUNKS = 80
E_PAD = NUM_TILES * C_CHUNKS * CHUNK
NBUF = 4
LOOKAHEAD = 2
PT = NPAD // 16
D1 = 64
D2 = 16
BR = 1024

_MESH = dict(core_axis_name="c", subcore_axis_name="s")


def _sc_deg(dst_t):
    @functools.partial(
        pl.kernel,
        out_type=jax.ShapeDtypeStruct((2, NPAD), jnp.float32),
        mesh=plsc.VectorSubcoreMesh(**_MESH),
        scratch_types=[
            pltpu.VMEM((C_CHUNKS, CHUNK), jnp.int32),
            pltpu.VMEM((CHUNK,), jnp.float32),
            pltpu.VMEM((PT,), jnp.float32),
            pltpu.VMEM_SHARED((NPAD,), jnp.float32),
        ],
    )
    def deg_kernel(dst_hbm, out_hbm, didx, ones, zbuf, acc):
        cid = lax.axis_index("c")
        sid = lax.axis_index("s")
        wid = sid * 2 + cid

        def fill_ones(i, _):
            ones[pl.ds(i * 16, 16)] = jnp.ones((16,), jnp.float32)
            return 0

        lax.fori_loop(0, CHUNK // 16, fill_ones, 0)

        def fill_zero(i, _):
            zbuf[pl.ds(i * 16, 16)] = jnp.zeros((16,), jnp.float32)
            return 0

        lax.fori_loop(0, PT // 16, fill_zero, 0)
        pltpu.sync_copy(zbuf, acc.at[pl.ds(sid * PT, PT)])
        plsc.subcore_barrier()

        pltpu.sync_copy(dst_hbm.at[wid], didx)

        def body(j, _):
            pltpu.sync_copy(ones, acc.at[didx.at[j]], add=True)
            return 0

        lax.fori_loop(0, C_CHUNKS, body, 0)
        plsc.subcore_barrier()
        pltpu.sync_copy(acc.at[pl.ds(sid * PT, PT)],
                        out_hbm.at[cid, pl.ds(sid * PT, PT)])

    return deg_kernel(dst_t)


def _sc_scatter(z, src_t, dst_t, d):
    @functools.partial(
        pl.kernel,
        out_type=jax.ShapeDtypeStruct((2, NPAD, d), jnp.float32),
        mesh=plsc.VectorSubcoreMesh(**_MESH),
        compiler_params=pltpu.CompilerParams(use_tc_tiling_on_sc=False),
        scratch_types=[
            pltpu.VMEM((C_CHUNKS, CHUNK), jnp.int32),
            pltpu.VMEM((C_CHUNKS, CHUNK), jnp.int32),
            pltpu.VMEM((CHUNK, d), jnp.float32),
            pltpu.VMEM((CHUNK, d), jnp.float32),
            pltpu.VMEM_SHARED((NPAD, d), jnp.float32),
            pltpu.SemaphoreType.DMA,
        ],
    )
    def scat_kernel(z_hbm, src_hbm, dst_hbm, out_hbm, sidx, didx, rows, zbuf,
                    acc, gsem):
        cid = lax.axis_index("c")
        sid = lax.axis_index("s")
        wid = sid * 2 + cid

        vecs_per_row = d // 16

        def fill_zero(i, _):
            r = i // vecs_per_row
            col = (i % vecs_per_row) * 16
            zbuf[r, pl.ds(col, 16)] = jnp.zeros((16,), jnp.float32)
            return 0

        lax.fori_loop(0, CHUNK * vecs_per_row, fill_zero, 0)

        def zero_acc(i, _):
            pltpu.sync_copy(zbuf, acc.at[pl.ds(sid * PT + i * CHUNK, CHUNK)])
            return 0

        lax.fori_loop(0, PT // CHUNK, zero_acc, 0)
        plsc.subcore_barrier()

        pltpu.sync_copy(src_hbm.at[wid], sidx)
        pltpu.sync_copy(dst_hbm.at[wid], didx)

        def body(j, _):
            pltpu.async_copy(z_hbm.at[sidx.at[j]], rows, gsem).wait()
            pltpu.sync_copy(rows, acc.at[didx.at[j]], add=True)
            return 0

        lax.fori_loop(0, C_CHUNKS, body, 0)
        plsc.subcore_barrier()

        def writeback(i, _):
            sl = pl.ds(sid * PT + i * CHUNK, CHUNK)
            pltpu.sync_copy(acc.at[sl], out_hbm.at[cid, sl])
            return 0

        lax.fori_loop(0, PT // CHUNK, writeback, 0)

    return scat_kernel(z, src_t, dst_t)


def _dinv_block(degp_ref):
    deg = degp_ref[0, :] + degp_ref[1, :] + 1.0
    return lax.rsqrt(deg)


def _tc1_body(x_ref, w_ref, degp_ref, z_ref):
    dinv = _dinv_block(degp_ref)
    xw = jnp.dot(x_ref[...], w_ref[...], preferred_element_type=jnp.float32)
    z_ref[...] = xw * dinv[:, None]


def _tc2_body(s1_ref, z1_ref, degp_ref, b1_ref, w2_ref, z2_ref):
    dinv = _dinv_block(degp_ref)
    s = s1_ref[0] + s1_ref[1] + z1_ref[...]
    h = jnp.maximum(s * dinv[:, None] + b1_ref[...], 0.0)
    z2_ref[...] = jnp.dot(h, w2_ref[...],
                          preferred_element_type=jnp.float32) * dinv[:, None]


def _tc3_body(s2_ref, z2_ref, degp_ref, b2_ref, o_ref):
    dinv = _dinv_block(degp_ref)
    o_ref[...] = (s2_ref[0] + s2_ref[1] + z2_ref[...]) * dinv[:, None] + b2_ref[...]


def _tc1(xpad, w1, degp):
    return pl.pallas_call(
        _tc1_body,
        grid=(NPAD // BR,),
        in_specs=[
            pl.BlockSpec((BR, 128), lambda i: (i, 0)),
            pl.BlockSpec((128, D1), lambda i: (0, 0)),
            pl.BlockSpec((2, BR), lambda i: (0, i)),
        ],
        out_specs=pl.BlockSpec((BR, D1), lambda i: (i, 0)),
        out_shape=jax.ShapeDtypeStruct((NPAD, D1), jnp.float32),
    )(xpad, w1, degp)


def _tc2(s1, z1, degp, b1, w2p):
    return pl.pallas_call(
        _tc2_body,
        grid=(NPAD // BR,),
        in_specs=[
            pl.BlockSpec((2, BR, D1), lambda i: (0, i, 0)),
            pl.BlockSpec((BR, D1), lambda i: (i, 0)),
            pl.BlockSpec((2, BR), lambda i: (0, i)),
            pl.BlockSpec((1, D1), lambda i: (0, 0)),
            pl.BlockSpec((D1, D2), lambda i: (0, 0)),
        ],
        out_specs=pl.BlockSpec((BR, D2), lambda i: (i, 0)),
        out_shape=jax.ShapeDtypeStruct((NPAD, D2), jnp.float32),
    )(s1, z1, degp, b1, w2p)


def _tc3(s2, z2, degp, b2p):
    return pl.pallas_call(
        _tc3_body,
        grid=(NPAD // BR,),
        in_specs=[
            pl.BlockSpec((2, BR, D2), lambda i: (0, i, 0)),
            pl.BlockSpec((BR, D2), lambda i: (i, 0)),
            pl.BlockSpec((2, BR), lambda i: (0, i)),
            pl.BlockSpec((1, D2), lambda i: (0, 0)),
        ],
        out_specs=pl.BlockSpec((BR, D2), lambda i: (i, 0)),
        out_shape=jax.ShapeDtypeStruct((NPAD, D2), jnp.float32),
    )(s2, z2, degp, b2p)


def kernel(x, edge_index, W1, b1, W2, b2):
    src = edge_index[0]
    dst = edge_index[1]
    pad = E_PAD - E_EDGES
    src_t = jnp.concatenate(
        [src, jnp.zeros((pad,), jnp.int32)]).reshape(NUM_TILES, C_CHUNKS, CHUNK)
    dummy = N_NODES + (jnp.arange(pad, dtype=jnp.int32) % (NPAD - N_NODES))
    dst_t = jnp.concatenate([dst, dummy]).reshape(NUM_TILES, C_CHUNKS, CHUNK)

    degp = _sc_deg(dst_t)
    xpad = jnp.pad(x, ((0, NPAD - N_NODES), (0, 0)))
    z1 = _tc1(xpad, W1, degp)
    s1 = _sc_scatter(z1, src_t, dst_t, D1)
    w2p = jnp.pad(W2, ((0, 0), (0, D2 - W2.shape[1])))
    z2 = _tc2(s1, z1, degp, b1.reshape(1, D1), w2p)
    s2 = _sc_scatter(z2, src_t, dst_t, D2)
    b2p = jnp.pad(b2, (0, D2 - b2.shape[0])).reshape(1, D2)
    outp = _tc3(s2, z2, degp, b2p)
    return outp[:N_NODES, :2]


# speedup vs baseline: 1.6658x; 1.6658x over previous
import functools

import jax
import jax.numpy as jnp
from jax import lax
from jax.experimental import pallas as pl
from jax.experimental.pallas import tpu as pltpu
from jax.experimental.pallas import tpu_sc as plsc

N_NODES = 10000
NPAD = 10240
E_EDGES = 320000
NUM_TILES = 32
CHUNK = 128
C_CHUNKS = 79
E_PAD = NUM_TILES * C_CHUNKS * CHUNK
NBUF = 4
LOOKAHEAD = 2
PT = NPAD // 16
D1 = 64
D2 = 16
BR = 1024

_MESH = dict(core_axis_name="c", subcore_axis_name="s")


def _sc_deg(dst_t):
    @functools.partial(
        pl.kernel,
        out_type=jax.ShapeDtypeStruct((2, NPAD), jnp.float32),
        mesh=plsc.VectorSubcoreMesh(**_MESH),
        scratch_types=[
            pltpu.VMEM((C_CHUNKS, CHUNK), jnp.int32),
            pltpu.VMEM((CHUNK,), jnp.float32),
            pltpu.VMEM((PT,), jnp.float32),
            pltpu.VMEM_SHARED((NPAD,), jnp.float32),
        ],
    )
    def deg_kernel(dst_hbm, out_hbm, didx, ones, zbuf, acc):
        cid = lax.axis_index("c")
        sid = lax.axis_index("s")
        wid = sid * 2 + cid

        def fill_ones(i, _):
            ones[pl.ds(i * 16, 16)] = jnp.ones((16,), jnp.float32)
            return 0

        lax.fori_loop(0, CHUNK // 16, fill_ones, 0)

        def fill_zero(i, _):
            zbuf[pl.ds(i * 16, 16)] = jnp.zeros((16,), jnp.float32)
            return 0

        lax.fori_loop(0, PT // 16, fill_zero, 0)
        pltpu.sync_copy(zbuf, acc.at[pl.ds(sid * PT, PT)])
        plsc.subcore_barrier()

        pltpu.sync_copy(dst_hbm.at[wid], didx)

        def body(j, _):
            pltpu.sync_copy(ones, acc.at[didx.at[j]], add=True)
            return 0

        lax.fori_loop(0, C_CHUNKS, body, 0)
        plsc.subcore_barrier()
        pltpu.sync_copy(acc.at[pl.ds(sid * PT, PT)],
                        out_hbm.at[cid, pl.ds(sid * PT, PT)])

    return deg_kernel(dst_t)


def _sc_scatter(z, src_t, dst_t, d):
    @functools.partial(
        pl.kernel,
        out_type=jax.ShapeDtypeStruct((2, NPAD, d), jnp.float32),
        mesh=plsc.VectorSubcoreMesh(**_MESH),
        compiler_params=pltpu.CompilerParams(use_tc_tiling_on_sc=False),
        scratch_types=[
            pltpu.VMEM((C_CHUNKS, CHUNK), jnp.int32),
            pltpu.VMEM((C_CHUNKS, CHUNK), jnp.int32),
            pltpu.VMEM((NBUF, CHUNK, d), jnp.float32),
            pltpu.VMEM((CHUNK, d), jnp.float32),
            pltpu.VMEM_SHARED((NPAD, d), jnp.float32),
            pltpu.SemaphoreType.DMA,
            pltpu.SemaphoreType.DMA,
        ],
    )
    def scat_kernel(z_hbm, src_hbm, dst_hbm, out_hbm, sidx, didx, rows, zbuf,
                    acc, gsem, ssem):
        cid = lax.axis_index("c")
        sid = lax.axis_index("s")
        wid = sid * 2 + cid

        vecs_per_row = d // 16

        def fill_zero(i, _):
            r = i // vecs_per_row
            col = (i % vecs_per_row) * 16
            zbuf[r, pl.ds(col, 16)] = jnp.zeros((16,), jnp.float32)
            return 0

        lax.fori_loop(0, CHUNK * vecs_per_row, fill_zero, 0)

        def zero_acc(i, _):
            pltpu.sync_copy(zbuf, acc.at[pl.ds(sid * PT + i * CHUNK, CHUNK)])
            return 0

        lax.fori_loop(0, PT // CHUNK, zero_acc, 0)
        plsc.subcore_barrier()

        pltpu.sync_copy(src_hbm.at[wid], sidx)
        pltpu.sync_copy(dst_hbm.at[wid], didx)

        def start_gather(j, b):
            pltpu.async_copy(z_hbm.at[sidx.at[j]], rows.at[b], gsem)

        def wait_gather():
            pltpu.make_async_copy(z_hbm.at[sidx.at[0]], rows.at[0], gsem).wait()

        def start_scatter(j, b):
            pltpu.async_copy(rows.at[b], acc.at[didx.at[j]], ssem, add=True)

        def wait_scatter():
            pltpu.make_async_copy(rows.at[0], acc.at[didx.at[0]], ssem).wait()

        for b in range(LOOKAHEAD):
            start_gather(b, b)

        def body(j, _):
            @pl.when(j >= LOOKAHEAD)
            def _():
                wait_scatter()

            @pl.when(j + LOOKAHEAD < C_CHUNKS)
            def _():
                start_gather(j + LOOKAHEAD, lax.rem(j + LOOKAHEAD, NBUF))

            wait_gather()
            start_scatter(j, lax.rem(j, NBUF))
            return 0

        lax.fori_loop(0, C_CHUNKS, body, 0)
        for _ in range(LOOKAHEAD):
            wait_scatter()
        plsc.subcore_barrier()

        def writeback(i, _):
            sl = pl.ds(sid * PT + i * CHUNK, CHUNK)
            pltpu.sync_copy(acc.at[sl], out_hbm.at[cid, sl])
            return 0

        lax.fori_loop(0, PT // CHUNK, writeback, 0)

    return scat_kernel(z, src_t, dst_t)


def _dinv_block(degp_ref):
    deg = degp_ref[0, :] + degp_ref[1, :] + 1.0
    return lax.rsqrt(deg)


def _tc1_body(x_ref, w_ref, degp_ref, z_ref):
    dinv = _dinv_block(degp_ref)
    xw = jnp.dot(x_ref[...], w_ref[...], preferred_element_type=jnp.float32)
    z_ref[...] = xw * dinv[:, None]


def _tc2_body(s1_ref, z1_ref, degp_ref, b1_ref, w2_ref, z2_ref):
    dinv = _dinv_block(degp_ref)
    s = s1_ref[0] + s1_ref[1] + z1_ref[...]
    h = jnp.maximum(s * dinv[:, None] + b1_ref[...], 0.0)
    z2_ref[...] = jnp.dot(h, w2_ref[...],
                          preferred_element_type=jnp.float32) * dinv[:, None]


def _tc3_body(s2_ref, z2_ref, degp_ref, b2_ref, o_ref):
    dinv = _dinv_block(degp_ref)
    o_ref[...] = (s2_ref[0] + s2_ref[1] + z2_ref[...]) * dinv[:, None] + b2_ref[...]


def _tc1(xpad, w1, degp):
    return pl.pallas_call(
        _tc1_body,
        grid=(NPAD // BR,),
        in_specs=[
            pl.BlockSpec((BR, 128), lambda i: (i, 0)),
            pl.BlockSpec((128, D1), lambda i: (0, 0)),
            pl.BlockSpec((2, BR), lambda i: (0, i)),
        ],
        out_specs=pl.BlockSpec((BR, D1), lambda i: (i, 0)),
        out_shape=jax.ShapeDtypeStruct((NPAD, D1), jnp.float32),
    )(xpad, w1, degp)


def _tc2(s1, z1, degp, b1, w2p):
    return pl.pallas_call(
        _tc2_body,
        grid=(NPAD // BR,),
        in_specs=[
            pl.BlockSpec((2, BR, D1), lambda i: (0, i, 0)),
            pl.BlockSpec((BR, D1), lambda i: (i, 0)),
            pl.BlockSpec((2, BR), lambda i: (0, i)),
            pl.BlockSpec((1, D1), lambda i: (0, 0)),
            pl.BlockSpec((D1, D2), lambda i: (0, 0)),
        ],
        out_specs=pl.BlockSpec((BR, D2), lambda i: (i, 0)),
        out_shape=jax.ShapeDtypeStruct((NPAD, D2), jnp.float32),
    )(s1, z1, degp, b1, w2p)


def _tc3(s2, z2, degp, b2p):
    return pl.pallas_call(
        _tc3_body,
        grid=(NPAD // BR,),
        in_specs=[
            pl.BlockSpec((2, BR, D2), lambda i: (0, i, 0)),
            pl.BlockSpec((BR, D2), lambda i: (i, 0)),
            pl.BlockSpec((2, BR), lambda i: (0, i)),
            pl.BlockSpec((1, D2), lambda i: (0, 0)),
        ],
        out_specs=pl.BlockSpec((BR, D2), lambda i: (i, 0)),
        out_shape=jax.ShapeDtypeStruct((NPAD, D2), jnp.float32),
    )(s2, z2, degp, b2p)


def kernel(x, edge_index, W1, b1, W2, b2):
    src = edge_index[0]
    dst = edge_index[1]
    pad = E_PAD - E_EDGES
    src_t = jnp.concatenate(
        [src, jnp.zeros((pad,), jnp.int32)]).reshape(NUM_TILES, C_CHUNKS, CHUNK)
    dummy = N_NODES + (jnp.arange(pad, dtype=jnp.int32) % (NPAD - N_NODES))
    dst_t = jnp.concatenate([dst, dummy]).reshape(NUM_TILES, C_CHUNKS, CHUNK)

    degp = _sc_deg(dst_t)
    xpad = jnp.pad(x, ((0, NPAD - N_NODES), (0, 0)))
    z1 = _tc1(xpad, W1, degp)
    s1 = _sc_scatter(z1, src_t, dst_t, D1)
    w2p = jnp.pad(W2, ((0, 0), (0, D2 - W2.shape[1])))
    z2 = _tc2(s1, z1, degp, b1.reshape(1, D1), w2p)
    s2 = _sc_scatter(z2, src_t, dst_t, D2)
    b2p = jnp.pad(b2, (0, D2 - b2.shape[0])).reshape(1, D2)
    outp = _tc3(s2, z2, degp, b2p)
    return outp[:N_NODES, :2]


# speedup vs baseline: 1.6809x; 1.0091x over previous
import functools

import jax
import jax.numpy as jnp
from jax import lax
from jax.experimental import pallas as pl
from jax.experimental.pallas import tpu as pltpu
from jax.experimental.pallas import tpu_sc as plsc

N_NODES = 10000
NPAD = 10240
E_EDGES = 320000
NUM_TILES = 32
CHUNK = 128
C_CHUNKS = 79
E_PAD = NUM_TILES * C_CHUNKS * CHUNK
NBUF = 6
LOOKAHEAD = 3
PT = NPAD // 16
D1 = 64
D2 = 16
BR = 1024

_MESH = dict(core_axis_name="c", subcore_axis_name="s")


def _sc_deg(dst_t):
    @functools.partial(
        pl.kernel,
        out_type=jax.ShapeDtypeStruct((2, NPAD), jnp.float32),
        mesh=plsc.VectorSubcoreMesh(**_MESH),
        scratch_types=[
            pltpu.VMEM((C_CHUNKS, CHUNK), jnp.int32),
            pltpu.VMEM((CHUNK,), jnp.float32),
            pltpu.VMEM((PT,), jnp.float32),
            pltpu.VMEM_SHARED((NPAD,), jnp.float32),
        ],
    )
    def deg_kernel(dst_hbm, out_hbm, didx, ones, zbuf, acc):
        cid = lax.axis_index("c")
        sid = lax.axis_index("s")
        wid = sid * 2 + cid

        def fill_ones(i, _):
            ones[pl.ds(i * 16, 16)] = jnp.ones((16,), jnp.float32)
            return 0

        lax.fori_loop(0, CHUNK // 16, fill_ones, 0)

        def fill_zero(i, _):
            zbuf[pl.ds(i * 16, 16)] = jnp.zeros((16,), jnp.float32)
            return 0

        lax.fori_loop(0, PT // 16, fill_zero, 0)
        pltpu.sync_copy(zbuf, acc.at[pl.ds(sid * PT, PT)])
        plsc.subcore_barrier()

        pltpu.sync_copy(dst_hbm.at[wid], didx)

        def body(j, _):
            pltpu.sync_copy(ones, acc.at[didx.at[j]], add=True)
            return 0

        lax.fori_loop(0, C_CHUNKS, body, 0)
        plsc.subcore_barrier()
        pltpu.sync_copy(acc.at[pl.ds(sid * PT, PT)],
                        out_hbm.at[cid, pl.ds(sid * PT, PT)])

    return deg_kernel(dst_t)


def _sc_scatter(z, src_t, dst_t, d):
    @functools.partial(
        pl.kernel,
        out_type=jax.ShapeDtypeStruct((2, NPAD, d), jnp.float32),
        mesh=plsc.VectorSubcoreMesh(**_MESH),
        compiler_params=pltpu.CompilerParams(use_tc_tiling_on_sc=False),
        scratch_types=[
            pltpu.VMEM((C_CHUNKS, CHUNK), jnp.int32),
            pltpu.VMEM((C_CHUNKS, CHUNK), jnp.int32),
            pltpu.VMEM((NBUF, CHUNK, d), jnp.float32),
            pltpu.VMEM((CHUNK, d), jnp.float32),
            pltpu.VMEM_SHARED((NPAD, d), jnp.float32),
            pltpu.SemaphoreType.DMA,
            pltpu.SemaphoreType.DMA,
        ],
    )
    def scat_kernel(z_hbm, src_hbm, dst_hbm, out_hbm, sidx, didx, rows, zbuf,
                    acc, gsem, ssem):
        cid = lax.axis_index("c")
        sid = lax.axis_index("s")
        wid = sid * 2 + cid

        vecs_per_row = d // 16

        def fill_zero(i, _):
            r = i // vecs_per_row
            col = (i % vecs_per_row) * 16
            zbuf[r, pl.ds(col, 16)] = jnp.zeros((16,), jnp.float32)
            return 0

        lax.fori_loop(0, CHUNK * vecs_per_row, fill_zero, 0)

        def zero_acc(i, _):
            pltpu.sync_copy(zbuf, acc.at[pl.ds(sid * PT + i * CHUNK, CHUNK)])
            return 0

        lax.fori_loop(0, PT // CHUNK, zero_acc, 0)
        plsc.subcore_barrier()

        pltpu.sync_copy(src_hbm.at[wid], sidx)
        pltpu.sync_copy(dst_hbm.at[wid], didx)

        def start_gather(j, b):
            pltpu.async_copy(z_hbm.at[sidx.at[j]], rows.at[b], gsem)

        def wait_gather():
            pltpu.make_async_copy(z_hbm.at[sidx.at[0]], rows.at[0], gsem).wait()

        def start_scatter(j, b):
            pltpu.async_copy(rows.at[b], acc.at[didx.at[j]], ssem, add=True)

        def wait_scatter():
            pltpu.make_async_copy(rows.at[0], acc.at[didx.at[0]], ssem).wait()

        for b in range(LOOKAHEAD):
            start_gather(b, b)

        def body(j, _):
            @pl.when(j >= LOOKAHEAD)
            def _():
                wait_scatter()

            @pl.when(j + LOOKAHEAD < C_CHUNKS)
            def _():
                start_gather(j + LOOKAHEAD, lax.rem(j + LOOKAHEAD, NBUF))

            wait_gather()
            start_scatter(j, lax.rem(j, NBUF))
            return 0

        lax.fori_loop(0, C_CHUNKS, body, 0)
        for _ in range(LOOKAHEAD):
            wait_scatter()
        plsc.subcore_barrier()

        def writeback(i, _):
            sl = pl.ds(sid * PT + i * CHUNK, CHUNK)
            pltpu.sync_copy(acc.at[sl], out_hbm.at[cid, sl])
            return 0

        lax.fori_loop(0, PT // CHUNK, writeback, 0)

    return scat_kernel(z, src_t, dst_t)


def _dinv_block(degp_ref):
    deg = degp_ref[0, :] + degp_ref[1, :] + 1.0
    return lax.rsqrt(deg)


def _tc1_body(x_ref, w_ref, degp_ref, z_ref):
    dinv = _dinv_block(degp_ref)
    xw = jnp.dot(x_ref[...], w_ref[...], preferred_element_type=jnp.float32)
    z_ref[...] = xw * dinv[:, None]


def _tc2_body(s1_ref, z1_ref, degp_ref, b1_ref, w2_ref, z2_ref):
    dinv = _dinv_block(degp_ref)
    s = s1_ref[0] + s1_ref[1] + z1_ref[...]
    h = jnp.maximum(s * dinv[:, None] + b1_ref[...], 0.0)
    z2_ref[...] = jnp.dot(h, w2_ref[...],
                          preferred_element_type=jnp.float32) * dinv[:, None]


def _tc3_body(s2_ref, z2_ref, degp_ref, b2_ref, o_ref):
    dinv = _dinv_block(degp_ref)
    o_ref[...] = (s2_ref[0] + s2_ref[1] + z2_ref[...]) * dinv[:, None] + b2_ref[...]


def _tc1(xpad, w1, degp):
    return pl.pallas_call(
        _tc1_body,
        grid=(NPAD // BR,),
        in_specs=[
            pl.BlockSpec((BR, 128), lambda i: (i, 0)),
            pl.BlockSpec((128, D1), lambda i: (0, 0)),
            pl.BlockSpec((2, BR), lambda i: (0, i)),
        ],
        out_specs=pl.BlockSpec((BR, D1), lambda i: (i, 0)),
        out_shape=jax.ShapeDtypeStruct((NPAD, D1), jnp.float32),
    )(xpad, w1, degp)


def _tc2(s1, z1, degp, b1, w2p):
    return pl.pallas_call(
        _tc2_body,
        grid=(NPAD // BR,),
        in_specs=[
            pl.BlockSpec((2, BR, D1), lambda i: (0, i, 0)),
            pl.BlockSpec((BR, D1), lambda i: (i, 0)),
            pl.BlockSpec((2, BR), lambda i: (0, i)),
            pl.BlockSpec((1, D1), lambda i: (0, 0)),
            pl.BlockSpec((D1, D2), lambda i: (0, 0)),
        ],
        out_specs=pl.BlockSpec((BR, D2), lambda i: (i, 0)),
        out_shape=jax.ShapeDtypeStruct((NPAD, D2), jnp.float32),
    )(s1, z1, degp, b1, w2p)


def _tc3(s2, z2, degp, b2p):
    return pl.pallas_call(
        _tc3_body,
        grid=(NPAD // BR,),
        in_specs=[
            pl.BlockSpec((2, BR, D2), lambda i: (0, i, 0)),
            pl.BlockSpec((BR, D2), lambda i: (i, 0)),
            pl.BlockSpec((2, BR), lambda i: (0, i)),
            pl.BlockSpec((1, D2), lambda i: (0, 0)),
        ],
        out_specs=pl.BlockSpec((BR, D2), lambda i: (i, 0)),
        out_shape=jax.ShapeDtypeStruct((NPAD, D2), jnp.float32),
    )(s2, z2, degp, b2p)


def kernel(x, edge_index, W1, b1, W2, b2):
    src = edge_index[0]
    dst = edge_index[1]
    pad = E_PAD - E_EDGES
    src_t = jnp.concatenate(
        [src, jnp.zeros((pad,), jnp.int32)]).reshape(NUM_TILES, C_CHUNKS, CHUNK)
    dummy = N_NODES + (jnp.arange(pad, dtype=jnp.int32) % (NPAD - N_NODES))
    dst_t = jnp.concatenate([dst, dummy]).reshape(NUM_TILES, C_CHUNKS, CHUNK)

    degp = _sc_deg(dst_t)
    xpad = jnp.pad(x, ((0, NPAD - N_NODES), (0, 0)))
    z1 = _tc1(xpad, W1, degp)
    s1 = _sc_scatter(z1, src_t, dst_t, D1)
    w2p = jnp.pad(W2, ((0, 0), (0, D2 - W2.shape[1])))
    z2 = _tc2(s1, z1, degp, b1.reshape(1, D1), w2p)
    s2 = _sc_scatter(z2, src_t, dst_t, D2)
    b2p = jnp.pad(b2, (0, D2 - b2.shape[0])).reshape(1, D2)
    outp = _tc3(s2, z2, degp, b2p)
    return outp[:N_NODES, :2]


# speedup vs baseline: 1.8065x; 1.0747x over previous
import functools

import jax
import jax.numpy as jnp
from jax import lax
from jax.experimental import pallas as pl
from jax.experimental.pallas import tpu as pltpu
from jax.experimental.pallas import tpu_sc as plsc

N_NODES = 10000
NPAD = 10240
E_EDGES = 320000
NUM_TILES = 32
CHUNK = 128
C_CHUNKS = 79
E_PAD = NUM_TILES * C_CHUNKS * CHUNK
NBUF = 6
LOOKAHEAD = 3
PT = NPAD // 16
D1 = 64
D2 = 16
BR = 1024

_MESH = dict(core_axis_name="c", subcore_axis_name="s")


def _sc_deg(dst_t):
    @functools.partial(
        pl.kernel,
        out_type=jax.ShapeDtypeStruct((2, NPAD), jnp.float32),
        mesh=plsc.VectorSubcoreMesh(**_MESH),
        scratch_types=[
            pltpu.VMEM((C_CHUNKS, CHUNK), jnp.int32),
            pltpu.VMEM((CHUNK,), jnp.float32),
            pltpu.VMEM((PT,), jnp.float32),
            pltpu.VMEM_SHARED((NPAD,), jnp.float32),
        ],
    )
    def deg_kernel(dst_hbm, out_hbm, didx, ones, zbuf, acc):
        cid = lax.axis_index("c")
        sid = lax.axis_index("s")
        wid = sid * 2 + cid

        def fill_ones(i, _):
            ones[pl.ds(i * 16, 16)] = jnp.ones((16,), jnp.float32)
            return 0

        lax.fori_loop(0, CHUNK // 16, fill_ones, 0)

        def fill_zero(i, _):
            zbuf[pl.ds(i * 16, 16)] = jnp.zeros((16,), jnp.float32)
            return 0

        lax.fori_loop(0, PT // 16, fill_zero, 0)
        pltpu.sync_copy(zbuf, acc.at[pl.ds(sid * PT, PT)])
        plsc.subcore_barrier()

        pltpu.sync_copy(dst_hbm.at[wid], didx)

        def body(j, _):
            pltpu.sync_copy(ones, acc.at[didx.at[j]], add=True)
            return 0

        lax.fori_loop(0, C_CHUNKS, body, 0)
        plsc.subcore_barrier()
        pltpu.sync_copy(acc.at[pl.ds(sid * PT, PT)],
                        out_hbm.at[cid, pl.ds(sid * PT, PT)])

    return deg_kernel(dst_t)


def _sc_scatter(z, src_t, dst_t, d):
    @functools.partial(
        pl.kernel,
        out_type=jax.ShapeDtypeStruct((2, NPAD, d), jnp.float32),
        mesh=plsc.VectorSubcoreMesh(**_MESH),
        compiler_params=pltpu.CompilerParams(use_tc_tiling_on_sc=False),
        scratch_types=[
            pltpu.VMEM((C_CHUNKS, CHUNK), jnp.int32),
            pltpu.VMEM((C_CHUNKS, CHUNK), jnp.int32),
            pltpu.VMEM((NBUF, CHUNK, d), jnp.float32),
            pltpu.VMEM((CHUNK, d), jnp.float32),
            pltpu.VMEM_SHARED((NPAD, d), jnp.float32),
            pltpu.VMEM_SHARED((NPAD, d if d <= 16 else 1), jnp.float32),
            pltpu.SemaphoreType.DMA,
            pltpu.SemaphoreType.DMA,
        ],
    )
    def scat_kernel(z_hbm, src_hbm, dst_hbm, out_hbm, sidx, didx, rows, zbuf,
                    acc, ztab, gsem, ssem):
        cid = lax.axis_index("c")
        sid = lax.axis_index("s")
        wid = sid * 2 + cid

        vecs_per_row = d // 16

        def fill_zero(i, _):
            r = i // vecs_per_row
            col = (i % vecs_per_row) * 16
            zbuf[r, pl.ds(col, 16)] = jnp.zeros((16,), jnp.float32)
            return 0

        lax.fori_loop(0, CHUNK * vecs_per_row, fill_zero, 0)

        def zero_acc(i, _):
            pltpu.sync_copy(zbuf, acc.at[pl.ds(sid * PT + i * CHUNK, CHUNK)])
            return 0

        lax.fori_loop(0, PT // CHUNK, zero_acc, 0)
        if d <= 16:
            zsl = pl.ds(sid * PT, PT)
            pltpu.sync_copy(z_hbm.at[zsl], ztab.at[zsl])
        plsc.subcore_barrier()

        pltpu.sync_copy(src_hbm.at[wid], sidx)
        pltpu.sync_copy(dst_hbm.at[wid], didx)

        ztbl = ztab if d <= 16 else z_hbm

        def start_gather(j, b):
            pltpu.async_copy(ztbl.at[sidx.at[j]], rows.at[b], gsem)

        def wait_gather():
            pltpu.make_async_copy(ztbl.at[sidx.at[0]], rows.at[0], gsem).wait()

        def start_scatter(j, b):
            pltpu.async_copy(rows.at[b], acc.at[didx.at[j]], ssem, add=True)

        def wait_scatter():
            pltpu.make_async_copy(rows.at[0], acc.at[didx.at[0]], ssem).wait()

        for b in range(LOOKAHEAD):
            start_gather(b, b)

        def body(j, _):
            @pl.when(j >= LOOKAHEAD)
            def _():
                wait_scatter()

            @pl.when(j + LOOKAHEAD < C_CHUNKS)
            def _():
                start_gather(j + LOOKAHEAD, lax.rem(j + LOOKAHEAD, NBUF))

            wait_gather()
            start_scatter(j, lax.rem(j, NBUF))
            return 0

        lax.fori_loop(0, C_CHUNKS, body, 0)
        for _ in range(LOOKAHEAD):
            wait_scatter()
        plsc.subcore_barrier()

        def writeback(i, _):
            sl = pl.ds(sid * PT + i * CHUNK, CHUNK)
            pltpu.sync_copy(acc.at[sl], out_hbm.at[cid, sl])
            return 0

        lax.fori_loop(0, PT // CHUNK, writeback, 0)

    return scat_kernel(z, src_t, dst_t)


def _dinv_block(degp_ref):
    deg = degp_ref[0, :] + degp_ref[1, :] + 1.0
    return lax.rsqrt(deg)


def _tc1_body(x_ref, w_ref, degp_ref, z_ref):
    dinv = _dinv_block(degp_ref)
    xw = jnp.dot(x_ref[...], w_ref[...], preferred_element_type=jnp.float32)
    z_ref[...] = xw * dinv[:, None]


def _tc2_body(s1_ref, z1_ref, degp_ref, b1_ref, w2_ref, z2_ref):
    dinv = _dinv_block(degp_ref)
    s = s1_ref[0] + s1_ref[1] + z1_ref[...]
    h = jnp.maximum(s * dinv[:, None] + b1_ref[...], 0.0)
    z2_ref[...] = jnp.dot(h, w2_ref[...],
                          preferred_element_type=jnp.float32) * dinv[:, None]


def _tc3_body(s2_ref, z2_ref, degp_ref, b2_ref, o_ref):
    dinv = _dinv_block(degp_ref)
    o_ref[...] = (s2_ref[0] + s2_ref[1] + z2_ref[...]) * dinv[:, None] + b2_ref[...]


def _tc1(xpad, w1, degp):
    return pl.pallas_call(
        _tc1_body,
        grid=(NPAD // BR,),
        in_specs=[
            pl.BlockSpec((BR, 128), lambda i: (i, 0)),
            pl.BlockSpec((128, D1), lambda i: (0, 0)),
            pl.BlockSpec((2, BR), lambda i: (0, i)),
        ],
        out_specs=pl.BlockSpec((BR, D1), lambda i: (i, 0)),
        out_shape=jax.ShapeDtypeStruct((NPAD, D1), jnp.float32),
    )(xpad, w1, degp)


def _tc2(s1, z1, degp, b1, w2p):
    return pl.pallas_call(
        _tc2_body,
        grid=(NPAD // BR,),
        in_specs=[
            pl.BlockSpec((2, BR, D1), lambda i: (0, i, 0)),
            pl.BlockSpec((BR, D1), lambda i: (i, 0)),
            pl.BlockSpec((2, BR), lambda i: (0, i)),
            pl.BlockSpec((1, D1), lambda i: (0, 0)),
            pl.BlockSpec((D1, D2), lambda i: (0, 0)),
        ],
        out_specs=pl.BlockSpec((BR, D2), lambda i: (i, 0)),
        out_shape=jax.ShapeDtypeStruct((NPAD, D2), jnp.float32),
    )(s1, z1, degp, b1, w2p)


def _tc3(s2, z2, degp, b2p):
    return pl.pallas_call(
        _tc3_body,
        grid=(NPAD // BR,),
        in_specs=[
            pl.BlockSpec((2, BR, D2), lambda i: (0, i, 0)),
            pl.BlockSpec((BR, D2), lambda i: (i, 0)),
            pl.BlockSpec((2, BR), lambda i: (0, i)),
            pl.BlockSpec((1, D2), lambda i: (0, 0)),
        ],
        out_specs=pl.BlockSpec((BR, D2), lambda i: (i, 0)),
        out_shape=jax.ShapeDtypeStruct((NPAD, D2), jnp.float32),
    )(s2, z2, degp, b2p)


def kernel(x, edge_index, W1, b1, W2, b2):
    src = edge_index[0]
    dst = edge_index[1]
    pad = E_PAD - E_EDGES
    src_t = jnp.concatenate(
        [src, jnp.zeros((pad,), jnp.int32)]).reshape(NUM_TILES, C_CHUNKS, CHUNK)
    dummy = N_NODES + (jnp.arange(pad, dtype=jnp.int32) % (NPAD - N_NODES))
    dst_t = jnp.concatenate([dst, dummy]).reshape(NUM_TILES, C_CHUNKS, CHUNK)

    degp = _sc_deg(dst_t)
    xpad = jnp.pad(x, ((0, NPAD - N_NODES), (0, 0)))
    z1 = _tc1(xpad, W1, degp)
    s1 = _sc_scatter(z1, src_t, dst_t, D1)
    w2p = jnp.pad(W2, ((0, 0), (0, D2 - W2.shape[1])))
    z2 = _tc2(s1, z1, degp, b1.reshape(1, D1), w2p)
    s2 = _sc_scatter(z2, src_t, dst_t, D2)
    b2p = jnp.pad(b2, (0, D2 - b2.shape[0])).reshape(1, D2)
    outp = _tc3(s2, z2, degp, b2p)
    return outp[:N_NODES, :2]


# speedup vs baseline: 2.5106x; 1.3897x over previous
import functools

import jax
import jax.numpy as jnp
from jax import lax
from jax.experimental import pallas as pl
from jax.experimental.pallas import tpu as pltpu
from jax.experimental.pallas import tpu_sc as plsc

N_NODES = 10000
NPAD = 10240
E_EDGES = 320000
NUM_TILES = 32
CHUNK = 128
C_CHUNKS = 79
E_PAD = NUM_TILES * C_CHUNKS * CHUNK
NBUF = 6
LOOKAHEAD = 3
PT = NPAD // 16
D1 = 64
D2 = 16
BR = 1024

_MESH = dict(core_axis_name="c", subcore_axis_name="s")


def _sc_deg(dst_t):
    @functools.partial(
        pl.kernel,
        out_type=jax.ShapeDtypeStruct((2, NPAD), jnp.float32),
        mesh=plsc.VectorSubcoreMesh(**_MESH),
        scratch_types=[
            pltpu.VMEM((C_CHUNKS, CHUNK), jnp.int32),
            pltpu.VMEM((CHUNK,), jnp.float32),
            pltpu.VMEM((PT,), jnp.float32),
            pltpu.VMEM_SHARED((NPAD,), jnp.float32),
        ],
    )
    def deg_kernel(dst_hbm, out_hbm, didx, ones, zbuf, acc):
        cid = lax.axis_index("c")
        sid = lax.axis_index("s")
        wid = sid * 2 + cid

        def fill_ones(i, _):
            ones[pl.ds(i * 16, 16)] = jnp.ones((16,), jnp.float32)
            return 0

        lax.fori_loop(0, CHUNK // 16, fill_ones, 0)

        def fill_zero(i, _):
            zbuf[pl.ds(i * 16, 16)] = jnp.zeros((16,), jnp.float32)
            return 0

        lax.fori_loop(0, PT // 16, fill_zero, 0)
        pltpu.sync_copy(zbuf, acc.at[pl.ds(sid * PT, PT)])
        plsc.subcore_barrier()

        pltpu.sync_copy(dst_hbm.at[wid], didx)

        def body(j, _):
            pltpu.sync_copy(ones, acc.at[didx.at[j]], add=True)
            return 0

        lax.fori_loop(0, C_CHUNKS, body, 0)
        plsc.subcore_barrier()
        pltpu.sync_copy(acc.at[pl.ds(sid * PT, PT)],
                        out_hbm.at[cid, pl.ds(sid * PT, PT)])

    return deg_kernel(dst_t)


def _sc_scatter(z, src_t, dst_t, d):
    dp = min(d, 32)
    phases = d // dp

    @functools.partial(
        pl.kernel,
        out_type=jax.ShapeDtypeStruct((2, NPAD, d), jnp.float32),
        mesh=plsc.VectorSubcoreMesh(**_MESH),
        compiler_params=pltpu.CompilerParams(use_tc_tiling_on_sc=False),
        scratch_types=[
            pltpu.VMEM((C_CHUNKS, CHUNK), jnp.int32),
            pltpu.VMEM((C_CHUNKS, CHUNK), jnp.int32),
            pltpu.VMEM((NBUF, CHUNK, dp), jnp.float32),
            pltpu.VMEM((CHUNK, dp), jnp.float32),
            pltpu.VMEM_SHARED((NPAD, dp), jnp.float32),
            pltpu.VMEM_SHARED((NPAD, dp), jnp.float32),
            pltpu.SemaphoreType.DMA,
            pltpu.SemaphoreType.DMA,
        ],
    )
    def scat_kernel(z_hbm, src_hbm, dst_hbm, out_hbm, sidx, didx, rows, zbuf,
                    acc, ztab, gsem, ssem):
        cid = lax.axis_index("c")
        sid = lax.axis_index("s")
        wid = sid * 2 + cid

        vecs_per_row = dp // 16

        def fill_zero(i, _):
            r = i // vecs_per_row
            col = (i % vecs_per_row) * 16
            zbuf[r, pl.ds(col, 16)] = jnp.zeros((16,), jnp.float32)
            return 0

        lax.fori_loop(0, CHUNK * vecs_per_row, fill_zero, 0)

        pltpu.sync_copy(src_hbm.at[wid], sidx)
        pltpu.sync_copy(dst_hbm.at[wid], didx)

        zsl = pl.ds(sid * PT, PT)

        def start_gather(j, b):
            pltpu.async_copy(ztab.at[sidx.at[j]], rows.at[b], gsem)

        def wait_gather():
            pltpu.make_async_copy(ztab.at[sidx.at[0]], rows.at[0], gsem).wait()

        def start_scatter(j, b):
            pltpu.async_copy(rows.at[b], acc.at[didx.at[j]], ssem, add=True)

        def wait_scatter():
            pltpu.make_async_copy(rows.at[0], acc.at[didx.at[0]], ssem).wait()

        for p in range(phases):
            csl = pl.ds(p * dp, dp)

            def zero_acc(i, _):
                pltpu.sync_copy(zbuf, acc.at[pl.ds(sid * PT + i * CHUNK, CHUNK)])
                return 0

            lax.fori_loop(0, PT // CHUNK, zero_acc, 0)
            if phases == 1:
                pltpu.sync_copy(z_hbm.at[zsl], ztab.at[zsl])
            else:
                pltpu.sync_copy(z_hbm.at[zsl, csl], ztab.at[zsl])
            plsc.subcore_barrier()

            for b in range(LOOKAHEAD):
                start_gather(b, b)

            def body(j, _):
                @pl.when(j >= LOOKAHEAD)
                def _():
                    wait_scatter()

                @pl.when(j + LOOKAHEAD < C_CHUNKS)
                def _():
                    start_gather(j + LOOKAHEAD, lax.rem(j + LOOKAHEAD, NBUF))

                wait_gather()
                start_scatter(j, lax.rem(j, NBUF))
                return 0

            lax.fori_loop(0, C_CHUNKS, body, 0)
            for _ in range(LOOKAHEAD):
                wait_scatter()
            plsc.subcore_barrier()

            def writeback(i, _):
                sl = pl.ds(sid * PT + i * CHUNK, CHUNK)
                if phases == 1:
                    pltpu.sync_copy(acc.at[sl], out_hbm.at[cid, sl])
                else:
                    pltpu.sync_copy(acc.at[sl], out_hbm.at[cid, sl, csl])
                return 0

            lax.fori_loop(0, PT // CHUNK, writeback, 0)
            if p + 1 < phases:
                plsc.subcore_barrier()

    return scat_kernel(z, src_t, dst_t)


def _dinv_block(degp_ref):
    deg = degp_ref[0, :] + degp_ref[1, :] + 1.0
    return lax.rsqrt(deg)


def _tc1_body(x_ref, w_ref, degp_ref, z_ref):
    dinv = _dinv_block(degp_ref)
    xw = jnp.dot(x_ref[...], w_ref[...], preferred_element_type=jnp.float32)
    z_ref[...] = xw * dinv[:, None]


def _tc2_body(s1_ref, z1_ref, degp_ref, b1_ref, w2_ref, z2_ref):
    dinv = _dinv_block(degp_ref)
    s = s1_ref[0] + s1_ref[1] + z1_ref[...]
    h = jnp.maximum(s * dinv[:, None] + b1_ref[...], 0.0)
    z2_ref[...] = jnp.dot(h, w2_ref[...],
                          preferred_element_type=jnp.float32) * dinv[:, None]


def _tc3_body(s2_ref, z2_ref, degp_ref, b2_ref, o_ref):
    dinv = _dinv_block(degp_ref)
    o_ref[...] = (s2_ref[0] + s2_ref[1] + z2_ref[...]) * dinv[:, None] + b2_ref[...]


def _tc1(xpad, w1, degp):
    return pl.pallas_call(
        _tc1_body,
        grid=(NPAD // BR,),
        in_specs=[
            pl.BlockSpec((BR, 128), lambda i: (i, 0)),
            pl.BlockSpec((128, D1), lambda i: (0, 0)),
            pl.BlockSpec((2, BR), lambda i: (0, i)),
        ],
        out_specs=pl.BlockSpec((BR, D1), lambda i: (i, 0)),
        out_shape=jax.ShapeDtypeStruct((NPAD, D1), jnp.float32),
    )(xpad, w1, degp)


def _tc2(s1, z1, degp, b1, w2p):
    return pl.pallas_call(
        _tc2_body,
        grid=(NPAD // BR,),
        in_specs=[
            pl.BlockSpec((2, BR, D1), lambda i: (0, i, 0)),
            pl.BlockSpec((BR, D1), lambda i: (i, 0)),
            pl.BlockSpec((2, BR), lambda i: (0, i)),
            pl.BlockSpec((1, D1), lambda i: (0, 0)),
            pl.BlockSpec((D1, D2), lambda i: (0, 0)),
        ],
        out_specs=pl.BlockSpec((BR, D2), lambda i: (i, 0)),
        out_shape=jax.ShapeDtypeStruct((NPAD, D2), jnp.float32),
    )(s1, z1, degp, b1, w2p)


def _tc3(s2, z2, degp, b2p):
    return pl.pallas_call(
        _tc3_body,
        grid=(NPAD // BR,),
        in_specs=[
            pl.BlockSpec((2, BR, D2), lambda i: (0, i, 0)),
            pl.BlockSpec((BR, D2), lambda i: (i, 0)),
            pl.BlockSpec((2, BR), lambda i: (0, i)),
            pl.BlockSpec((1, D2), lambda i: (0, 0)),
        ],
        out_specs=pl.BlockSpec((BR, D2), lambda i: (i, 0)),
        out_shape=jax.ShapeDtypeStruct((NPAD, D2), jnp.float32),
    )(s2, z2, degp, b2p)


def kernel(x, edge_index, W1, b1, W2, b2):
    src = edge_index[0]
    dst = edge_index[1]
    pad = E_PAD - E_EDGES
    src_t = jnp.concatenate(
        [src, jnp.zeros((pad,), jnp.int32)]).reshape(NUM_TILES, C_CHUNKS, CHUNK)
    dummy = N_NODES + (jnp.arange(pad, dtype=jnp.int32) % (NPAD - N_NODES))
    dst_t = jnp.concatenate([dst, dummy]).reshape(NUM_TILES, C_CHUNKS, CHUNK)

    degp = _sc_deg(dst_t)
    xpad = jnp.pad(x, ((0, NPAD - N_NODES), (0, 0)))
    z1 = _tc1(xpad, W1, degp)
    s1 = _sc_scatter(z1, src_t, dst_t, D1)
    w2p = jnp.pad(W2, ((0, 0), (0, D2 - W2.shape[1])))
    z2 = _tc2(s1, z1, degp, b1.reshape(1, D1), w2p)
    s2 = _sc_scatter(z2, src_t, dst_t, D2)
    b2p = jnp.pad(b2, (0, D2 - b2.shape[0])).reshape(1, D2)
    outp = _tc3(s2, z2, degp, b2p)
    return outp[:N_NODES, :2]


# speedup vs baseline: 2.5993x; 1.0354x over previous
import functools

import jax
import jax.numpy as jnp
from jax import lax
from jax.experimental import pallas as pl
from jax.experimental.pallas import tpu as pltpu
from jax.experimental.pallas import tpu_sc as plsc

N_NODES = 10000
NPAD = 10240
E_EDGES = 320000
NUM_TILES = 32
CHUNK = 128
C_CHUNKS = 79
E_PAD = NUM_TILES * C_CHUNKS * CHUNK
NBUF = 6
LOOKAHEAD = 3
PT = NPAD // 16
D1 = 64
D2 = 16
BR = 1024

_MESH = dict(core_axis_name="c", subcore_axis_name="s")


def _sc_deg(dst_t):
    @functools.partial(
        pl.kernel,
        out_type=jax.ShapeDtypeStruct((2, NPAD), jnp.float32),
        mesh=plsc.VectorSubcoreMesh(**_MESH),
        scratch_types=[
            pltpu.VMEM((C_CHUNKS, CHUNK), jnp.int32),
            pltpu.VMEM((CHUNK,), jnp.float32),
            pltpu.VMEM((PT,), jnp.float32),
            pltpu.VMEM_SHARED((NPAD,), jnp.float32),
            pltpu.SemaphoreType.DMA,
        ],
    )
    def deg_kernel(dst_hbm, out_hbm, didx, ones, zbuf, acc, dsem):
        cid = lax.axis_index("c")
        sid = lax.axis_index("s")
        wid = sid * 2 + cid

        def fill_ones(i, _):
            ones[pl.ds(i * 16, 16)] = jnp.ones((16,), jnp.float32)
            return 0

        lax.fori_loop(0, CHUNK // 16, fill_ones, 0)

        def fill_zero(i, _):
            zbuf[pl.ds(i * 16, 16)] = jnp.zeros((16,), jnp.float32)
            return 0

        lax.fori_loop(0, PT // 16, fill_zero, 0)
        pltpu.sync_copy(zbuf, acc.at[pl.ds(sid * PT, PT)])
        plsc.subcore_barrier()

        pltpu.sync_copy(dst_hbm.at[wid], didx)

        def wait_one():
            pltpu.make_async_copy(ones, acc.at[didx.at[0]], dsem).wait()

        def body(j, _):
            @pl.when(j >= 8)
            def _():
                wait_one()

            pltpu.async_copy(ones, acc.at[didx.at[j]], dsem, add=True)
            return 0

        lax.fori_loop(0, C_CHUNKS, body, 0)
        for _ in range(8):
            wait_one()
        plsc.subcore_barrier()
        pltpu.sync_copy(acc.at[pl.ds(sid * PT, PT)],
                        out_hbm.at[cid, pl.ds(sid * PT, PT)])

    return deg_kernel(dst_t)


def _sc_scatter(z, src_t, dst_t, d):
    dp = min(d, 32)
    phases = d // dp

    @functools.partial(
        pl.kernel,
        out_type=jax.ShapeDtypeStruct((2, NPAD, d), jnp.float32),
        mesh=plsc.VectorSubcoreMesh(**_MESH),
        compiler_params=pltpu.CompilerParams(use_tc_tiling_on_sc=False),
        scratch_types=[
            pltpu.VMEM((C_CHUNKS, CHUNK), jnp.int32),
            pltpu.VMEM((C_CHUNKS, CHUNK), jnp.int32),
            pltpu.VMEM((NBUF, CHUNK, dp), jnp.float32),
            pltpu.VMEM((CHUNK, dp), jnp.float32),
            pltpu.VMEM_SHARED((NPAD, dp), jnp.float32),
            pltpu.VMEM_SHARED((NPAD, dp), jnp.float32),
            pltpu.SemaphoreType.DMA,
            pltpu.SemaphoreType.DMA,
        ],
    )
    def scat_kernel(z_hbm, src_hbm, dst_hbm, out_hbm, sidx, didx, rows, zbuf,
                    acc, ztab, gsem, ssem):
        cid = lax.axis_index("c")
        sid = lax.axis_index("s")
        wid = sid * 2 + cid

        vecs_per_row = dp // 16

        def fill_zero(i, _):
            r = i // vecs_per_row
            col = (i % vecs_per_row) * 16
            zbuf[r, pl.ds(col, 16)] = jnp.zeros((16,), jnp.float32)
            return 0

        lax.fori_loop(0, CHUNK * vecs_per_row, fill_zero, 0)

        pltpu.sync_copy(src_hbm.at[wid], sidx)
        pltpu.sync_copy(dst_hbm.at[wid], didx)

        zsl = pl.ds(sid * PT, PT)

        def start_gather(j, b):
            pltpu.async_copy(ztab.at[sidx.at[j]], rows.at[b], gsem)

        def wait_gather():
            pltpu.make_async_copy(ztab.at[sidx.at[0]], rows.at[0], gsem).wait()

        def start_scatter(j, b):
            pltpu.async_copy(rows.at[b], acc.at[didx.at[j]], ssem, add=True)

        def wait_scatter():
            pltpu.make_async_copy(rows.at[0], acc.at[didx.at[0]], ssem).wait()

        for p in range(phases):
            csl = pl.ds(p * dp, dp)

            def zero_acc(i, _):
                pltpu.sync_copy(zbuf, acc.at[pl.ds(sid * PT + i * CHUNK, CHUNK)])
                return 0

            lax.fori_loop(0, PT // CHUNK, zero_acc, 0)
            if phases == 1:
                pltpu.sync_copy(z_hbm.at[zsl], ztab.at[zsl])
            else:
                pltpu.sync_copy(z_hbm.at[zsl, csl], ztab.at[zsl])
            plsc.subcore_barrier()

            for b in range(LOOKAHEAD):
                start_gather(b, b)

            def body(j, _):
                @pl.when(j >= LOOKAHEAD)
                def _():
                    wait_scatter()

                @pl.when(j + LOOKAHEAD < C_CHUNKS)
                def _():
                    start_gather(j + LOOKAHEAD, lax.rem(j + LOOKAHEAD, NBUF))

                wait_gather()
                start_scatter(j, lax.rem(j, NBUF))
                return 0

            lax.fori_loop(0, C_CHUNKS, body, 0)
            for _ in range(LOOKAHEAD):
                wait_scatter()
            plsc.subcore_barrier()

            def writeback(i, _):
                sl = pl.ds(sid * PT + i * CHUNK, CHUNK)
                if phases == 1:
                    pltpu.sync_copy(acc.at[sl], out_hbm.at[cid, sl])
                else:
                    pltpu.sync_copy(acc.at[sl], out_hbm.at[cid, sl, csl])
                return 0

            lax.fori_loop(0, PT // CHUNK, writeback, 0)
            if p + 1 < phases:
                plsc.subcore_barrier()

    return scat_kernel(z, src_t, dst_t)


def _dinv_block(degp_ref):
    deg = degp_ref[0, :] + degp_ref[1, :] + 1.0
    return lax.rsqrt(deg)


def _tc1_body(x_ref, w_ref, degp_ref, z_ref):
    dinv = _dinv_block(degp_ref)
    xw = jnp.dot(x_ref[...], w_ref[...], preferred_element_type=jnp.float32)
    z_ref[...] = xw * dinv[:, None]


def _tc2_body(s1_ref, z1_ref, degp_ref, b1_ref, w2_ref, z2_ref):
    dinv = _dinv_block(degp_ref)
    s = s1_ref[0] + s1_ref[1] + z1_ref[...]
    h = jnp.maximum(s * dinv[:, None] + b1_ref[...], 0.0)
    z2_ref[...] = jnp.dot(h, w2_ref[...],
                          preferred_element_type=jnp.float32) * dinv[:, None]


def _tc3_body(s2_ref, z2_ref, degp_ref, b2_ref, o_ref):
    dinv = _dinv_block(degp_ref)
    o = (s2_ref[0] + s2_ref[1] + z2_ref[...]) * dinv[:, None] + b2_ref[...]
    o_ref[...] = o[:, :2]


def _tc1(x, w1, degp):
    return pl.pallas_call(
        _tc1_body,
        grid=(NPAD // BR,),
        in_specs=[
            pl.BlockSpec((BR, 128), lambda i: (i, 0)),
            pl.BlockSpec((128, D1), lambda i: (0, 0)),
            pl.BlockSpec((2, BR), lambda i: (0, i)),
        ],
        out_specs=pl.BlockSpec((BR, D1), lambda i: (i, 0)),
        out_shape=jax.ShapeDtypeStruct((NPAD, D1), jnp.float32),
    )(x, w1, degp)


def _tc2(s1, z1, degp, b1, w2p):
    return pl.pallas_call(
        _tc2_body,
        grid=(NPAD // BR,),
        in_specs=[
            pl.BlockSpec((2, BR, D1), lambda i: (0, i, 0)),
            pl.BlockSpec((BR, D1), lambda i: (i, 0)),
            pl.BlockSpec((2, BR), lambda i: (0, i)),
            pl.BlockSpec((1, D1), lambda i: (0, 0)),
            pl.BlockSpec((D1, D2), lambda i: (0, 0)),
        ],
        out_specs=pl.BlockSpec((BR, D2), lambda i: (i, 0)),
        out_shape=jax.ShapeDtypeStruct((NPAD, D2), jnp.float32),
    )(s1, z1, degp, b1, w2p)


def _tc3(s2, z2, degp, b2p):
    return pl.pallas_call(
        _tc3_body,
        grid=(NPAD // BR,),
        in_specs=[
            pl.BlockSpec((2, BR, D2), lambda i: (0, i, 0)),
            pl.BlockSpec((BR, D2), lambda i: (i, 0)),
            pl.BlockSpec((2, BR), lambda i: (0, i)),
            pl.BlockSpec((1, D2), lambda i: (0, 0)),
        ],
        out_specs=pl.BlockSpec((BR, 2), lambda i: (i, 0)),
        out_shape=jax.ShapeDtypeStruct((N_NODES, 2), jnp.float32),
    )(s2, z2, degp, b2p)


def kernel(x, edge_index, W1, b1, W2, b2):
    src = edge_index[0]
    dst = edge_index[1]
    pad = E_PAD - E_EDGES
    src_t = jnp.concatenate(
        [src, jnp.zeros((pad,), jnp.int32)]).reshape(NUM_TILES, C_CHUNKS, CHUNK)
    dummy = N_NODES + (jnp.arange(pad, dtype=jnp.int32) % (NPAD - N_NODES))
    dst_t = jnp.concatenate([dst, dummy]).reshape(NUM_TILES, C_CHUNKS, CHUNK)

    degp = _sc_deg(dst_t)
    z1 = _tc1(x, W1, degp)
    s1 = _sc_scatter(z1, src_t, dst_t, D1)
    w2p = jnp.pad(W2, ((0, 0), (0, D2 - W2.shape[1])))
    z2 = _tc2(s1, z1, degp, b1.reshape(1, D1), w2p)
    s2 = _sc_scatter(z2, src_t, dst_t, D2)
    b2p = jnp.pad(b2, (0, D2 - b2.shape[0])).reshape(1, D2)
    return _tc3(s2, z2, degp, b2p)


# speedup vs baseline: 2.6275x; 1.0108x over previous
import functools

import jax
import jax.numpy as jnp
from jax import lax
from jax.experimental import pallas as pl
from jax.experimental.pallas import tpu as pltpu
from jax.experimental.pallas import tpu_sc as plsc

N_NODES = 10000
NPAD = 10240
E_EDGES = 320000
NUM_TILES = 32
CHUNK = 128
C_CHUNKS = 79
E_ROWS = E_EDGES // CHUNK
TAIL_CHUNKS = E_ROWS - 31 * C_CHUNKS
NBUF = 6
LOOKAHEAD = 3
PT = NPAD // 16
D1 = 64
D2 = 16
BR = 1024

_MESH = dict(core_axis_name="c", subcore_axis_name="s")


def _sc_deg(dst_t):
    @functools.partial(
        pl.kernel,
        out_type=jax.ShapeDtypeStruct((2, NPAD), jnp.float32),
        mesh=plsc.VectorSubcoreMesh(**_MESH),
        compiler_params=pltpu.CompilerParams(use_tc_tiling_on_sc=False),
        scratch_types=[
            pltpu.VMEM((C_CHUNKS, CHUNK), jnp.int32),
            pltpu.VMEM((CHUNK,), jnp.float32),
            pltpu.VMEM((PT,), jnp.float32),
            pltpu.VMEM_SHARED((NPAD,), jnp.float32),
            pltpu.SemaphoreType.DMA,
        ],
    )
    def deg_kernel(dst_hbm, out_hbm, didx, ones, zbuf, acc, dsem):
        cid = lax.axis_index("c")
        sid = lax.axis_index("s")
        wid = sid * 2 + cid

        def fill_ones(i, _):
            ones[pl.ds(i * 16, 16)] = jnp.ones((16,), jnp.float32)
            return 0

        lax.fori_loop(0, CHUNK // 16, fill_ones, 0)

        def fill_zero(i, _):
            zbuf[pl.ds(i * 16, 16)] = jnp.zeros((16,), jnp.float32)
            return 0

        lax.fori_loop(0, PT // 16, fill_zero, 0)
        pltpu.sync_copy(zbuf, acc.at[pl.ds(sid * PT, PT)])
        plsc.subcore_barrier()

        nc = jnp.where(wid == NUM_TILES - 1, TAIL_CHUNKS, C_CHUNKS)

        @pl.when(wid < NUM_TILES - 1)
        def _():
            pltpu.sync_copy(dst_hbm.at[pl.ds(C_CHUNKS * wid, C_CHUNKS)], didx)

        @pl.when(wid == NUM_TILES - 1)
        def _():
            pltpu.sync_copy(dst_hbm.at[pl.ds(C_CHUNKS * (NUM_TILES - 1),
                                             TAIL_CHUNKS)],
                            didx.at[pl.ds(0, TAIL_CHUNKS)])

        def wait_one():
            pltpu.make_async_copy(ones, acc.at[didx.at[0]], dsem).wait()

        def body(j, _):
            @pl.when(j >= 8)
            def _():
                wait_one()

            pltpu.async_copy(ones, acc.at[didx.at[j]], dsem, add=True)
            return 0

        lax.fori_loop(0, nc, body, 0)
        for _ in range(8):
            wait_one()
        plsc.subcore_barrier()
        pltpu.sync_copy(acc.at[pl.ds(sid * PT, PT)],
                        out_hbm.at[cid, pl.ds(sid * PT, PT)])

    return deg_kernel(dst_t)


def _sc_scatter(z, src_t, dst_t, d):
    dp = min(d, 32)
    phases = d // dp

    @functools.partial(
        pl.kernel,
        out_type=jax.ShapeDtypeStruct((2, NPAD, d), jnp.float32),
        mesh=plsc.VectorSubcoreMesh(**_MESH),
        compiler_params=pltpu.CompilerParams(use_tc_tiling_on_sc=False),
        scratch_types=[
            pltpu.VMEM((C_CHUNKS, CHUNK), jnp.int32),
            pltpu.VMEM((C_CHUNKS, CHUNK), jnp.int32),
            pltpu.VMEM((NBUF, CHUNK, dp), jnp.float32),
            pltpu.VMEM((CHUNK, dp), jnp.float32),
            pltpu.VMEM_SHARED((NPAD, dp), jnp.float32),
            pltpu.VMEM_SHARED((NPAD, dp), jnp.float32),
            pltpu.SemaphoreType.DMA,
            pltpu.SemaphoreType.DMA,
        ],
    )
    def scat_kernel(z_hbm, src_hbm, dst_hbm, out_hbm, sidx, didx, rows, zbuf,
                    acc, ztab, gsem, ssem):
        cid = lax.axis_index("c")
        sid = lax.axis_index("s")
        wid = sid * 2 + cid

        vecs_per_row = dp // 16

        def fill_zero(i, _):
            r = i // vecs_per_row
            col = (i % vecs_per_row) * 16
            zbuf[r, pl.ds(col, 16)] = jnp.zeros((16,), jnp.float32)
            return 0

        lax.fori_loop(0, CHUNK * vecs_per_row, fill_zero, 0)

        nc = jnp.where(wid == NUM_TILES - 1, TAIL_CHUNKS, C_CHUNKS)

        @pl.when(wid < NUM_TILES - 1)
        def _():
            row0 = pl.ds(C_CHUNKS * wid, C_CHUNKS)
            pltpu.sync_copy(src_hbm.at[row0], sidx)
            pltpu.sync_copy(dst_hbm.at[row0], didx)

        @pl.when(wid == NUM_TILES - 1)
        def _():
            row0 = pl.ds(C_CHUNKS * (NUM_TILES - 1), TAIL_CHUNKS)
            tsl = pl.ds(0, TAIL_CHUNKS)
            pltpu.sync_copy(src_hbm.at[row0], sidx.at[tsl])
            pltpu.sync_copy(dst_hbm.at[row0], didx.at[tsl])

        zsl = pl.ds(sid * PT, PT)

        def start_gather(j, b):
            pltpu.async_copy(ztab.at[sidx.at[j]], rows.at[b], gsem)

        def wait_gather():
            pltpu.make_async_copy(ztab.at[sidx.at[0]], rows.at[0], gsem).wait()

        def start_scatter(j, b):
            pltpu.async_copy(rows.at[b], acc.at[didx.at[j]], ssem, add=True)

        def wait_scatter():
            pltpu.make_async_copy(rows.at[0], acc.at[didx.at[0]], ssem).wait()

        for p in range(phases):
            csl = pl.ds(p * dp, dp)

            def zero_acc(i, _):
                pltpu.sync_copy(zbuf, acc.at[pl.ds(sid * PT + i * CHUNK, CHUNK)])
                return 0

            lax.fori_loop(0, PT // CHUNK, zero_acc, 0)
            if phases == 1:
                pltpu.sync_copy(z_hbm.at[zsl], ztab.at[zsl])
            else:
                pltpu.sync_copy(z_hbm.at[zsl, csl], ztab.at[zsl])
            plsc.subcore_barrier()

            for b in range(LOOKAHEAD):
                start_gather(b, b)

            def body(j, _):
                @pl.when(j >= LOOKAHEAD)
                def _():
                    wait_scatter()

                @pl.when(j + LOOKAHEAD < nc)
                def _():
                    start_gather(j + LOOKAHEAD, lax.rem(j + LOOKAHEAD, NBUF))

                wait_gather()
                start_scatter(j, lax.rem(j, NBUF))
                return 0

            lax.fori_loop(0, nc, body, 0)
            for _ in range(LOOKAHEAD):
                wait_scatter()
            plsc.subcore_barrier()

            def writeback(i, _):
                sl = pl.ds(sid * PT + i * CHUNK, CHUNK)
                if phases == 1:
                    pltpu.sync_copy(acc.at[sl], out_hbm.at[cid, sl])
                else:
                    pltpu.sync_copy(acc.at[sl], out_hbm.at[cid, sl, csl])
                return 0

            lax.fori_loop(0, PT // CHUNK, writeback, 0)
            if p + 1 < phases:
                plsc.subcore_barrier()

    return scat_kernel(z, src_t, dst_t)


def _dinv_block(degp_ref):
    deg = degp_ref[0, :] + degp_ref[1, :] + 1.0
    return lax.rsqrt(deg)


def _tc1_body(x_ref, w_ref, degp_ref, z_ref):
    dinv = _dinv_block(degp_ref)
    xw = jnp.dot(x_ref[...], w_ref[...], preferred_element_type=jnp.float32)
    z_ref[...] = xw * dinv[:, None]


def _tc2_body(s1_ref, z1_ref, degp_ref, b1_ref, w2_ref, z2_ref):
    dinv = _dinv_block(degp_ref)
    s = s1_ref[0] + s1_ref[1] + z1_ref[...]
    h = jnp.maximum(s * dinv[:, None] + b1_ref[...], 0.0)
    z2_ref[...] = jnp.dot(h, w2_ref[...],
                          preferred_element_type=jnp.float32) * dinv[:, None]


def _tc3_body(s2_ref, z2_ref, degp_ref, b2_ref, o_ref):
    dinv = _dinv_block(degp_ref)
    o = (s2_ref[0] + s2_ref[1] + z2_ref[...]) * dinv[:, None] + b2_ref[...]
    o_ref[...] = o[:, :2]


def _tc1(x, w1, degp):
    return pl.pallas_call(
        _tc1_body,
        grid=(NPAD // BR,),
        in_specs=[
            pl.BlockSpec((BR, 128), lambda i: (i, 0)),
            pl.BlockSpec((128, D1), lambda i: (0, 0)),
            pl.BlockSpec((2, BR), lambda i: (0, i)),
        ],
        out_specs=pl.BlockSpec((BR, D1), lambda i: (i, 0)),
        out_shape=jax.ShapeDtypeStruct((NPAD, D1), jnp.float32),
    )(x, w1, degp)


def _tc2(s1, z1, degp, b1, w2p):
    return pl.pallas_call(
        _tc2_body,
        grid=(NPAD // BR,),
        in_specs=[
            pl.BlockSpec((2, BR, D1), lambda i: (0, i, 0)),
            pl.BlockSpec((BR, D1), lambda i: (i, 0)),
            pl.BlockSpec((2, BR), lambda i: (0, i)),
            pl.BlockSpec((1, D1), lambda i: (0, 0)),
            pl.BlockSpec((D1, D2), lambda i: (0, 0)),
        ],
        out_specs=pl.BlockSpec((BR, D2), lambda i: (i, 0)),
        out_shape=jax.ShapeDtypeStruct((NPAD, D2), jnp.float32),
    )(s1, z1, degp, b1, w2p)


def _tc3(s2, z2, degp, b2p):
    return pl.pallas_call(
        _tc3_body,
        grid=(NPAD // BR,),
        in_specs=[
            pl.BlockSpec((2, BR, D2), lambda i: (0, i, 0)),
            pl.BlockSpec((BR, D2), lambda i: (i, 0)),
            pl.BlockSpec((2, BR), lambda i: (0, i)),
            pl.BlockSpec((1, D2), lambda i: (0, 0)),
        ],
        out_specs=pl.BlockSpec((BR, 2), lambda i: (i, 0)),
        out_shape=jax.ShapeDtypeStruct((N_NODES, 2), jnp.float32),
    )(s2, z2, degp, b2p)


def kernel(x, edge_index, W1, b1, W2, b2):
    src_t = edge_index[0].reshape(E_ROWS, CHUNK)
    dst_t = edge_index[1].reshape(E_ROWS, CHUNK)

    degp = _sc_deg(dst_t)
    z1 = _tc1(x, W1, degp)
    s1 = _sc_scatter(z1, src_t, dst_t, D1)
    w2p = jnp.pad(W2, ((0, 0), (0, D2 - W2.shape[1])))
    z2 = _tc2(s1, z1, degp, b1.reshape(1, D1), w2p)
    s2 = _sc_scatter(z2, src_t, dst_t, D2)
    b2p = jnp.pad(b2, (0, D2 - b2.shape[0])).reshape(1, D2)
    return _tc3(s2, z2, degp, b2p)


# speedup vs baseline: 2.9638x; 1.1280x over previous
import functools

import jax
import jax.numpy as jnp
from jax import lax
from jax.experimental import pallas as pl
from jax.experimental.pallas import tpu as pltpu
from jax.experimental.pallas import tpu_sc as plsc

N_NODES = 10000
NPAD = 10240
E_EDGES = 320000
NUM_TILES = 32
CHUNK = 128
C_CHUNKS = 79
E_ROWS = E_EDGES // CHUNK
TAIL_CHUNKS = E_ROWS - 31 * C_CHUNKS
NBUF = 6
LOOKAHEAD = 3
PT = NPAD // 16
D1 = 64
D2 = 16
BR = 1024

_MESH = dict(core_axis_name="c", subcore_axis_name="s")


def _sc_deg(dst_t):
    @functools.partial(
        pl.kernel,
        out_type=jax.ShapeDtypeStruct((2, NPAD), jnp.float32),
        mesh=plsc.VectorSubcoreMesh(**_MESH),
        compiler_params=pltpu.CompilerParams(use_tc_tiling_on_sc=False),
        scratch_types=[
            pltpu.VMEM((C_CHUNKS, CHUNK), jnp.int32),
            pltpu.VMEM((CHUNK,), jnp.float32),
            pltpu.VMEM((PT,), jnp.float32),
            pltpu.VMEM_SHARED((NPAD,), jnp.float32),
            pltpu.SemaphoreType.DMA,
        ],
    )
    def deg_kernel(dst_hbm, out_hbm, didx, ones, zbuf, acc, dsem):
        cid = lax.axis_index("c")
        sid = lax.axis_index("s")
        wid = sid * 2 + cid

        def fill_ones(i, _):
            ones[pl.ds(i * 16, 16)] = jnp.ones((16,), jnp.float32)
            return 0

        lax.fori_loop(0, CHUNK // 16, fill_ones, 0)

        def fill_zero(i, _):
            zbuf[pl.ds(i * 16, 16)] = jnp.zeros((16,), jnp.float32)
            return 0

        lax.fori_loop(0, PT // 16, fill_zero, 0)
        pltpu.sync_copy(zbuf, acc.at[pl.ds(sid * PT, PT)])
        plsc.subcore_barrier()

        nc = jnp.where(wid == NUM_TILES - 1, TAIL_CHUNKS, C_CHUNKS)

        @pl.when(wid < NUM_TILES - 1)
        def _():
            pltpu.sync_copy(dst_hbm.at[pl.ds(C_CHUNKS * wid, C_CHUNKS)], didx)

        @pl.when(wid == NUM_TILES - 1)
        def _():
            pltpu.sync_copy(dst_hbm.at[pl.ds(C_CHUNKS * (NUM_TILES - 1),
                                             TAIL_CHUNKS)],
                            didx.at[pl.ds(0, TAIL_CHUNKS)])

        def wait_one():
            pltpu.make_async_copy(ones, acc.at[didx.at[0]], dsem).wait()

        def body(j, _):
            @pl.when(j >= 8)
            def _():
                wait_one()

            pltpu.async_copy(ones, acc.at[didx.at[j]], dsem, add=True)
            return 0

        lax.fori_loop(0, nc, body, 0)
        for _ in range(8):
            wait_one()
        plsc.subcore_barrier()
        pltpu.sync_copy(acc.at[pl.ds(sid * PT, PT)],
                        out_hbm.at[cid, pl.ds(sid * PT, PT)])

    return deg_kernel(dst_t)


def _sc_scatter(z, src_t, dst_t, d):
    dp = min(d, 32)
    phases = d // dp

    @functools.partial(
        pl.kernel,
        out_type=jax.ShapeDtypeStruct((2, NPAD, 128), jnp.float32),
        mesh=plsc.VectorSubcoreMesh(**_MESH),
        compiler_params=pltpu.CompilerParams(use_tc_tiling_on_sc=False),
        scratch_types=[
            pltpu.VMEM((C_CHUNKS, CHUNK), jnp.int32),
            pltpu.VMEM((C_CHUNKS, CHUNK), jnp.int32),
            pltpu.VMEM((NBUF, CHUNK, dp), jnp.float32),
            pltpu.VMEM((CHUNK, dp), jnp.float32),
            pltpu.VMEM_SHARED((NPAD, dp), jnp.float32),
            pltpu.VMEM_SHARED((NPAD, dp), jnp.float32),
            pltpu.SemaphoreType.DMA,
            pltpu.SemaphoreType.DMA,
        ],
    )
    def scat_kernel(z_hbm, src_hbm, dst_hbm, out_hbm, sidx, didx, rows, zbuf,
                    acc, ztab, gsem, ssem):
        cid = lax.axis_index("c")
        sid = lax.axis_index("s")
        wid = sid * 2 + cid

        vecs_per_row = dp // 16

        def fill_zero(i, _):
            r = i // vecs_per_row
            col = (i % vecs_per_row) * 16
            zbuf[r, pl.ds(col, 16)] = jnp.zeros((16,), jnp.float32)
            return 0

        lax.fori_loop(0, CHUNK * vecs_per_row, fill_zero, 0)

        nc = jnp.where(wid == NUM_TILES - 1, TAIL_CHUNKS, C_CHUNKS)

        @pl.when(wid < NUM_TILES - 1)
        def _():
            row0 = pl.ds(C_CHUNKS * wid, C_CHUNKS)
            pltpu.sync_copy(src_hbm.at[row0], sidx)
            pltpu.sync_copy(dst_hbm.at[row0], didx)

        @pl.when(wid == NUM_TILES - 1)
        def _():
            row0 = pl.ds(C_CHUNKS * (NUM_TILES - 1), TAIL_CHUNKS)
            tsl = pl.ds(0, TAIL_CHUNKS)
            pltpu.sync_copy(src_hbm.at[row0], sidx.at[tsl])
            pltpu.sync_copy(dst_hbm.at[row0], didx.at[tsl])

        zsl = pl.ds(sid * PT, PT)

        def start_gather(j, b):
            pltpu.async_copy(ztab.at[sidx.at[j]], rows.at[b], gsem)

        def wait_gather():
            pltpu.make_async_copy(ztab.at[sidx.at[0]], rows.at[0], gsem).wait()

        def start_scatter(j, b):
            pltpu.async_copy(rows.at[b], acc.at[didx.at[j]], ssem, add=True)

        def wait_scatter():
            pltpu.make_async_copy(rows.at[0], acc.at[didx.at[0]], ssem).wait()

        for p in range(phases):
            csl = pl.ds(p * dp, dp)

            def zero_acc(i, _):
                pltpu.sync_copy(zbuf, acc.at[pl.ds(sid * PT + i * CHUNK, CHUNK)])
                return 0

            lax.fori_loop(0, PT // CHUNK, zero_acc, 0)
            pltpu.sync_copy(z_hbm.at[zsl, csl], ztab.at[zsl])
            plsc.subcore_barrier()

            for b in range(LOOKAHEAD):
                start_gather(b, b)

            def body(j, _):
                @pl.when(j >= LOOKAHEAD)
                def _():
                    wait_scatter()

                @pl.when(j + LOOKAHEAD < nc)
                def _():
                    start_gather(j + LOOKAHEAD, lax.rem(j + LOOKAHEAD, NBUF))

                wait_gather()
                start_scatter(j, lax.rem(j, NBUF))
                return 0

            lax.fori_loop(0, nc, body, 0)
            for _ in range(LOOKAHEAD):
                wait_scatter()
            plsc.subcore_barrier()

            def writeback(i, _):
                sl = pl.ds(sid * PT + i * CHUNK, CHUNK)
                pltpu.sync_copy(acc.at[sl], out_hbm.at[cid, sl, csl])
                return 0

            lax.fori_loop(0, PT // CHUNK, writeback, 0)
            if p + 1 < phases:
                plsc.subcore_barrier()

    return scat_kernel(z, src_t, dst_t)


def _dinv_block(degp_ref):
    deg = degp_ref[0, :] + degp_ref[1, :] + 1.0
    return lax.rsqrt(deg)


def _tc1_body(x_ref, w_ref, degp_ref, z_ref):
    dinv = _dinv_block(degp_ref)
    xw = jnp.dot(x_ref[...], w_ref[...], preferred_element_type=jnp.float32)
    z = xw * dinv[:, None]
    z_ref[...] = jnp.concatenate(
        [z, jnp.zeros((z.shape[0], 128 - D1), jnp.float32)], axis=1)


def _tc2_body(s1_ref, z1_ref, degp_ref, b1_ref, w2_ref, z2_ref):
    dinv = _dinv_block(degp_ref)
    s = s1_ref[0, :, :D1] + s1_ref[1, :, :D1] + z1_ref[:, :D1]
    h = jnp.maximum(s * dinv[:, None] + b1_ref[...], 0.0)
    z2 = jnp.dot(h, w2_ref[...],
                 preferred_element_type=jnp.float32) * dinv[:, None]
    z2_ref[...] = jnp.concatenate(
        [z2, jnp.zeros((z2.shape[0], 128 - D2), jnp.float32)], axis=1)


def _tc3_body(s2_ref, z2_ref, degp_ref, b2_ref, o_ref):
    dinv = _dinv_block(degp_ref)
    o = (s2_ref[0, :, :D2] + s2_ref[1, :, :D2]
         + z2_ref[:, :D2]) * dinv[:, None] + b2_ref[...]
    o_ref[...] = o[:, :2]


def _tc1(x, w1, degp):
    return pl.pallas_call(
        _tc1_body,
        grid=(NPAD // BR,),
        in_specs=[
            pl.BlockSpec((BR, 128), lambda i: (i, 0)),
            pl.BlockSpec((128, D1), lambda i: (0, 0)),
            pl.BlockSpec((2, BR), lambda i: (0, i)),
        ],
        out_specs=pl.BlockSpec((BR, 128), lambda i: (i, 0)),
        out_shape=jax.ShapeDtypeStruct((NPAD, 128), jnp.float32),
    )(x, w1, degp)


def _tc2(s1, z1, degp, b1, w2p):
    return pl.pallas_call(
        _tc2_body,
        grid=(NPAD // BR,),
        in_specs=[
            pl.BlockSpec((2, BR, 128), lambda i: (0, i, 0)),
            pl.BlockSpec((BR, 128), lambda i: (i, 0)),
            pl.BlockSpec((2, BR), lambda i: (0, i)),
            pl.BlockSpec((1, D1), lambda i: (0, 0)),
            pl.BlockSpec((D1, D2), lambda i: (0, 0)),
        ],
        out_specs=pl.BlockSpec((BR, 128), lambda i: (i, 0)),
        out_shape=jax.ShapeDtypeStruct((NPAD, 128), jnp.float32),
    )(s1, z1, degp, b1, w2p)


def _tc3(s2, z2, degp, b2p):
    return pl.pallas_call(
        _tc3_body,
        grid=(NPAD // BR,),
        in_specs=[
            pl.BlockSpec((2, BR, 128), lambda i: (0, i, 0)),
            pl.BlockSpec((BR, 128), lambda i: (i, 0)),
            pl.BlockSpec((2, BR), lambda i: (0, i)),
            pl.BlockSpec((1, D2), lambda i: (0, 0)),
        ],
        out_specs=pl.BlockSpec((BR, 2), lambda i: (i, 0)),
        out_shape=jax.ShapeDtypeStruct((N_NODES, 2), jnp.float32),
    )(s2, z2, degp, b2p)


def kernel(x, edge_index, W1, b1, W2, b2):
    src_t = edge_index[0].reshape(E_ROWS, CHUNK)
    dst_t = edge_index[1].reshape(E_ROWS, CHUNK)

    degp = _sc_deg(dst_t)
    z1 = _tc1(x, W1, degp)
    s1 = _sc_scatter(z1, src_t, dst_t, D1)
    w2p = jnp.pad(W2, ((0, 0), (0, D2 - W2.shape[1])))
    z2 = _tc2(s1, z1, degp, b1.reshape(1, D1), w2p)
    s2 = _sc_scatter(z2, src_t, dst_t, D2)
    b2p = jnp.pad(b2, (0, D2 - b2.shape[0])).reshape(1, D2)
    return _tc3(s2, z2, degp, b2p)


# speedup vs baseline: 2.9651x; 1.0004x over previous
import functools

import jax
import jax.numpy as jnp
from jax import lax
from jax.experimental import pallas as pl
from jax.experimental.pallas import tpu as pltpu
from jax.experimental.pallas import tpu_sc as plsc

N_NODES = 10000
NPAD = 10240
E_EDGES = 320000
NUM_TILES = 32
CHUNK = 128
C_CHUNKS = 79
E_ROWS = E_EDGES // CHUNK
TAIL_CHUNKS = E_ROWS - 31 * C_CHUNKS
NBUF = 8
LOOKAHEAD = 4
PT = NPAD // 16
D1 = 64
D2 = 16
BR = 1024

_MESH = dict(core_axis_name="c", subcore_axis_name="s")


def _sc_deg(dst_t):
    @functools.partial(
        pl.kernel,
        out_type=jax.ShapeDtypeStruct((2, NPAD), jnp.float32),
        mesh=plsc.VectorSubcoreMesh(**_MESH),
        compiler_params=pltpu.CompilerParams(use_tc_tiling_on_sc=False),
        scratch_types=[
            pltpu.VMEM((C_CHUNKS, CHUNK), jnp.int32),
            pltpu.VMEM((CHUNK,), jnp.float32),
            pltpu.VMEM((PT,), jnp.float32),
            pltpu.VMEM_SHARED((NPAD,), jnp.float32),
            pltpu.SemaphoreType.DMA,
        ],
    )
    def deg_kernel(dst_hbm, out_hbm, didx, ones, zbuf, acc, dsem):
        cid = lax.axis_index("c")
        sid = lax.axis_index("s")
        wid = sid * 2 + cid

        def fill_ones(i, _):
            ones[pl.ds(i * 16, 16)] = jnp.ones((16,), jnp.float32)
            return 0

        lax.fori_loop(0, CHUNK // 16, fill_ones, 0)

        def fill_zero(i, _):
            zbuf[pl.ds(i * 16, 16)] = jnp.zeros((16,), jnp.float32)
            return 0

        lax.fori_loop(0, PT // 16, fill_zero, 0)
        pltpu.sync_copy(zbuf, acc.at[pl.ds(sid * PT, PT)])
        plsc.subcore_barrier()

        nc = jnp.where(wid == NUM_TILES - 1, TAIL_CHUNKS, C_CHUNKS)

        @pl.when(wid < NUM_TILES - 1)
        def _():
            pltpu.sync_copy(dst_hbm.at[pl.ds(C_CHUNKS * wid, C_CHUNKS)], didx)

        @pl.when(wid == NUM_TILES - 1)
        def _():
            pltpu.sync_copy(dst_hbm.at[pl.ds(C_CHUNKS * (NUM_TILES - 1),
                                             TAIL_CHUNKS)],
                            didx.at[pl.ds(0, TAIL_CHUNKS)])

        def wait_one():
            pltpu.make_async_copy(ones, acc.at[didx.at[0]], dsem).wait()

        def body(j, _):
            @pl.when(j >= 8)
            def _():
                wait_one()

            pltpu.async_copy(ones, acc.at[didx.at[j]], dsem, add=True)
            return 0

        lax.fori_loop(0, nc, body, 0)
        for _ in range(8):
            wait_one()
        plsc.subcore_barrier()
        pltpu.sync_copy(acc.at[pl.ds(sid * PT, PT)],
                        out_hbm.at[cid, pl.ds(sid * PT, PT)])

    return deg_kernel(dst_t)


def _sc_scatter(z, src_t, dst_t, d):
    dp = min(d, 32)
    phases = d // dp

    @functools.partial(
        pl.kernel,
        out_type=jax.ShapeDtypeStruct((2, NPAD, 128), jnp.float32),
        mesh=plsc.VectorSubcoreMesh(**_MESH),
        compiler_params=pltpu.CompilerParams(use_tc_tiling_on_sc=False),
        scratch_types=[
            pltpu.VMEM((C_CHUNKS, CHUNK), jnp.int32),
            pltpu.VMEM((C_CHUNKS, CHUNK), jnp.int32),
            pltpu.VMEM((NBUF, CHUNK, dp), jnp.float32),
            pltpu.VMEM((CHUNK, dp), jnp.float32),
            pltpu.VMEM_SHARED((NPAD, dp), jnp.float32),
            pltpu.VMEM_SHARED((NPAD, dp), jnp.float32),
            pltpu.SemaphoreType.DMA,
            pltpu.SemaphoreType.DMA,
        ],
    )
    def scat_kernel(z_hbm, src_hbm, dst_hbm, out_hbm, sidx, didx, rows, zbuf,
                    acc, ztab, gsem, ssem):
        cid = lax.axis_index("c")
        sid = lax.axis_index("s")
        wid = sid * 2 + cid

        vecs_per_row = dp // 16

        def fill_zero(i, _):
            r = i // vecs_per_row
            col = (i % vecs_per_row) * 16
            zbuf[r, pl.ds(col, 16)] = jnp.zeros((16,), jnp.float32)
            return 0

        lax.fori_loop(0, CHUNK * vecs_per_row, fill_zero, 0)

        nc = jnp.where(wid == NUM_TILES - 1, TAIL_CHUNKS, C_CHUNKS)

        @pl.when(wid < NUM_TILES - 1)
        def _():
            row0 = pl.ds(C_CHUNKS * wid, C_CHUNKS)
            pltpu.sync_copy(src_hbm.at[row0], sidx)
            pltpu.sync_copy(dst_hbm.at[row0], didx)

        @pl.when(wid == NUM_TILES - 1)
        def _():
            row0 = pl.ds(C_CHUNKS * (NUM_TILES - 1), TAIL_CHUNKS)
            tsl = pl.ds(0, TAIL_CHUNKS)
            pltpu.sync_copy(src_hbm.at[row0], sidx.at[tsl])
            pltpu.sync_copy(dst_hbm.at[row0], didx.at[tsl])

        zsl = pl.ds(sid * PT, PT)

        def start_gather(j, b):
            pltpu.async_copy(ztab.at[sidx.at[j]], rows.at[b], gsem)

        def wait_gather():
            pltpu.make_async_copy(ztab.at[sidx.at[0]], rows.at[0], gsem).wait()

        def start_scatter(j, b):
            pltpu.async_copy(rows.at[b], acc.at[didx.at[j]], ssem, add=True)

        def wait_scatter():
            pltpu.make_async_copy(rows.at[0], acc.at[didx.at[0]], ssem).wait()

        for p in range(phases):
            csl = pl.ds(p * dp, dp)

            def zero_acc(i, _):
                pltpu.sync_copy(zbuf, acc.at[pl.ds(sid * PT + i * CHUNK, CHUNK)])
                return 0

            lax.fori_loop(0, PT // CHUNK, zero_acc, 0)
            pltpu.sync_copy(z_hbm.at[zsl, csl], ztab.at[zsl])
            plsc.subcore_barrier()

            for b in range(LOOKAHEAD):
                start_gather(b, b)

            def body(j, _):
                @pl.when(j >= LOOKAHEAD)
                def _():
                    wait_scatter()

                @pl.when(j + LOOKAHEAD < nc)
                def _():
                    start_gather(j + LOOKAHEAD, lax.rem(j + LOOKAHEAD, NBUF))

                wait_gather()
                start_scatter(j, lax.rem(j, NBUF))
                return 0

            lax.fori_loop(0, nc, body, 0)
            for _ in range(LOOKAHEAD):
                wait_scatter()
            plsc.subcore_barrier()

            def writeback(i, _):
                sl = pl.ds(sid * PT + i * CHUNK, CHUNK)
                pltpu.sync_copy(acc.at[sl], out_hbm.at[cid, sl, csl])
                return 0

            lax.fori_loop(0, PT // CHUNK, writeback, 0)
            if p + 1 < phases:
                plsc.subcore_barrier()

    return scat_kernel(z, src_t, dst_t)


def _dinv_block(degp_ref):
    deg = degp_ref[0, :] + degp_ref[1, :] + 1.0
    return lax.rsqrt(deg)


def _tc1_body(x_ref, w_ref, degp_ref, z_ref):
    dinv = _dinv_block(degp_ref)
    xw = jnp.dot(x_ref[...], w_ref[...], preferred_element_type=jnp.float32)
    z = xw * dinv[:, None]
    z_ref[...] = jnp.concatenate(
        [z, jnp.zeros((z.shape[0], 128 - D1), jnp.float32)], axis=1)


def _tc2_body(s1_ref, z1_ref, degp_ref, b1_ref, w2_ref, z2_ref):
    dinv = _dinv_block(degp_ref)
    s = s1_ref[0, :, :D1] + s1_ref[1, :, :D1] + z1_ref[:, :D1]
    h = jnp.maximum(s * dinv[:, None] + b1_ref[...], 0.0)
    z2 = jnp.dot(h, w2_ref[...],
                 preferred_element_type=jnp.float32) * dinv[:, None]
    z2_ref[...] = jnp.concatenate(
        [z2, jnp.zeros((z2.shape[0], 128 - D2), jnp.float32)], axis=1)


def _tc3_body(s2_ref, z2_ref, degp_ref, b2_ref, o_ref):
    dinv = _dinv_block(degp_ref)
    o = (s2_ref[0, :, :D2] + s2_ref[1, :, :D2]
         + z2_ref[:, :D2]) * dinv[:, None] + b2_ref[...]
    o_ref[...] = o[:, :2]


def _tc1(x, w1, degp):
    return pl.pallas_call(
        _tc1_body,
        grid=(NPAD // BR,),
        in_specs=[
            pl.BlockSpec((BR, 128), lambda i: (i, 0)),
            pl.BlockSpec((128, D1), lambda i: (0, 0)),
            pl.BlockSpec((2, BR), lambda i: (0, i)),
        ],
        out_specs=pl.BlockSpec((BR, 128), lambda i: (i, 0)),
        out_shape=jax.ShapeDtypeStruct((NPAD, 128), jnp.float32),
    )(x, w1, degp)


def _tc2(s1, z1, degp, b1, w2p):
    return pl.pallas_call(
        _tc2_body,
        grid=(NPAD // BR,),
        in_specs=[
            pl.BlockSpec((2, BR, 128), lambda i: (0, i, 0)),
            pl.BlockSpec((BR, 128), lambda i: (i, 0)),
            pl.BlockSpec((2, BR), lambda i: (0, i)),
            pl.BlockSpec((1, D1), lambda i: (0, 0)),
            pl.BlockSpec((D1, D2), lambda i: (0, 0)),
        ],
        out_specs=pl.BlockSpec((BR, 128), lambda i: (i, 0)),
        out_shape=jax.ShapeDtypeStruct((NPAD, 128), jnp.float32),
    )(s1, z1, degp, b1, w2p)


def _tc3(s2, z2, degp, b2p):
    return pl.pallas_call(
        _tc3_body,
        grid=(NPAD // BR,),
        in_specs=[
            pl.BlockSpec((2, BR, 128), lambda i: (0, i, 0)),
            pl.BlockSpec((BR, 128), lambda i: (i, 0)),
            pl.BlockSpec((2, BR), lambda i: (0, i)),
            pl.BlockSpec((1, D2), lambda i: (0, 0)),
        ],
        out_specs=pl.BlockSpec((BR, 2), lambda i: (i, 0)),
        out_shape=jax.ShapeDtypeStruct((N_NODES, 2), jnp.float32),
    )(s2, z2, degp, b2p)


def kernel(x, edge_index, W1, b1, W2, b2):
    src_t = edge_index[0].reshape(E_ROWS, CHUNK)
    dst_t = edge_index[1].reshape(E_ROWS, CHUNK)

    degp = _sc_deg(dst_t)
    z1 = _tc1(x, W1, degp)
    s1 = _sc_scatter(z1, src_t, dst_t, D1)
    w2p = jnp.pad(W2, ((0, 0), (0, D2 - W2.shape[1])))
    z2 = _tc2(s1, z1, degp, b1.reshape(1, D1), w2p)
    s2 = _sc_scatter(z2, src_t, dst_t, D2)
    b2p = jnp.pad(b2, (0, D2 - b2.shape[0])).reshape(1, D2)
    return _tc3(s2, z2, degp, b2p)
